# baseline (speedup 1.0000x reference)
import jax
import jax.numpy as jnp
from jax import lax
from jax.experimental import pallas as pl
from jax.experimental.pallas import tpu as pltpu
from jax.experimental.pallas import tpu_sc as plsc

N = 10000
E = 320000
IN_CH = 128
HID = 64

NC = 2
NS = 16
NW = NC * NS
CHUNK = 128
NCH = E // CHUNK
CPT = NCH // NW
TAIL = NCH - NW * CPT
NPAD = 10240
SLICE = NPAD // NS


def _enc_body(x_ref, w_ref, out_ref):
    out_ref[...] = jnp.dot(x_ref[...].astype(jnp.bfloat16),
                           w_ref[...].astype(jnp.bfloat16),
                           preferred_element_type=jnp.float32)


def _sc_body(gidx_ref, sidx_ref, y_ref, out_ref,
             gidx, sidx, rows0, rows1, stage, agg, sem0, sem1):
    c = lax.axis_index("c")
    s = lax.axis_index("s")
    w = s * NC + c
    zv = jnp.zeros((16,), jnp.float32)

    def zrow(i, carry):
        for jj in range(HID // 16):
            stage[i, pl.ds(jj * 16, 16)] = zv
        return carry

    lax.fori_loop(0, SLICE, zrow, 0)
    pltpu.sync_copy(stage, agg.at[pl.ds(s * SLICE, SLICE)])
    base = CPT * w + jnp.minimum(w, TAIL)
    pltpu.sync_copy(gidx_ref.at[pl.ds(base, CPT)], gidx.at[pl.ds(0, CPT)])
    pltpu.sync_copy(sidx_ref.at[pl.ds(base, CPT)], sidx.at[pl.ds(0, CPT)])

    @pl.when(w < TAIL)
    def _():
        pltpu.sync_copy(gidx_ref.at[pl.ds(base + CPT, 1)],
                        gidx.at[pl.ds(CPT, 1)])
        pltpu.sync_copy(sidx_ref.at[pl.ds(base + CPT, 1)],
                        sidx.at[pl.ds(CPT, 1)])

    plsc.subcore_barrier()

    pltpu.async_copy(y_ref.at[gidx.at[0]], rows0, sem0)

    def step(jj, carry):
        j0 = 2 * jj
        j1 = j0 + 1
        jn = jnp.minimum(j0 + 2, CPT - 2)
        pltpu.async_copy(y_ref.at[gidx.at[j1]], rows1, sem1)
        pltpu.make_async_copy(y_ref.at[gidx.at[j0]], rows0, sem0).wait()
        pltpu.sync_copy(rows0, agg.at[sidx.at[j0]], add=True)
        pltpu.async_copy(y_ref.at[gidx.at[jn]], rows0, sem0)
        pltpu.make_async_copy(y_ref.at[gidx.at[j1]], rows1, sem1).wait()
        pltpu.sync_copy(rows1, agg.at[sidx.at[j1]], add=True)
        return carry

    lax.fori_loop(0, CPT // 2, step, 0)
    pltpu.make_async_copy(y_ref.at[gidx.at[0]], rows0, sem0).wait()

    @pl.when(w < TAIL)
    def _():
        pltpu.async_copy(y_ref.at[gidx.at[CPT]], rows0, sem0)
        pltpu.make_async_copy(y_ref.at[gidx.at[CPT]], rows0, sem0).wait()
        pltpu.sync_copy(rows0, agg.at[sidx.at[CPT]], add=True)

    plsc.subcore_barrier()
    pltpu.sync_copy(agg.at[pl.ds(s * SLICE, SLICE)], stage)
    pltpu.sync_copy(stage, out_ref.at[c, pl.ds(s * SLICE, SLICE)])


_sc_scatter = pl.kernel(
    _sc_body,
    out_type=jax.ShapeDtypeStruct((NC, NPAD, HID), jnp.float32),
    mesh=plsc.VectorSubcoreMesh(core_axis_name="c", subcore_axis_name="s"),
    compiler_params=pltpu.CompilerParams(use_tc_tiling_on_sc=False),
    scratch_types=[
        pltpu.VMEM((CPT + 1, CHUNK), jnp.int32),
        pltpu.VMEM((CPT + 1, CHUNK), jnp.int32),
        pltpu.VMEM((CHUNK, HID), jnp.float32),
        pltpu.VMEM((CHUNK, HID), jnp.float32),
        pltpu.VMEM((SLICE, HID), jnp.float32),
        pltpu.VMEM_SHARED((NPAD, HID), jnp.float32),
        pltpu.SemaphoreType.DMA,
        pltpu.SemaphoreType.DMA,
    ],
)


def _dec_body(x_ref, p0_ref, p1_ref, wroot_ref, benc_ref,
              wdec_ref, bdec_ref, out_ref):
    agg = p0_ref[0] + p1_ref[0]
    h = (
        jnp.dot(x_ref[...].astype(jnp.bfloat16),
                wroot_ref[...].astype(jnp.bfloat16),
                preferred_element_type=jnp.float32)
        + agg + benc_ref[...]
    )
    h = jnp.maximum(h, 0.0)
    out_ref[...] = (
        jnp.dot(h.astype(jnp.bfloat16), wdec_ref[...].astype(jnp.bfloat16),
                preferred_element_type=jnp.float32)
        + bdec_ref[...]
    )


_ROWS_BLK = 1280


@jax.jit
def _forward(x, edge_index, W_root, W_rel, b_enc, W_dec, b_dec):
    gidx = edge_index[0].astype(jnp.int32).reshape(NCH, CHUNK)
    sidx = edge_index[1].astype(jnp.int32).reshape(NCH, CHUNK)

    y = pl.pallas_call(
        _enc_body,
        grid=(NS,),
        in_specs=[
            pl.BlockSpec((SLICE, IN_CH), lambda i: (i, 0)),
            pl.BlockSpec((IN_CH, HID), lambda i: (0, 0)),
        ],
        out_specs=pl.BlockSpec((SLICE, HID), lambda i: (i, 0)),
        out_shape=jax.ShapeDtypeStruct((NPAD, HID), jnp.float32),
    )(x, W_rel)

    parts = _sc_scatter(gidx, sidx, y)

    out = pl.pallas_call(
        _dec_body,
        grid=(NPAD // _ROWS_BLK,),
        in_specs=[
            pl.BlockSpec((_ROWS_BLK, IN_CH), lambda i: (i, 0)),
            pl.BlockSpec((1, _ROWS_BLK, HID), lambda i: (0, i, 0)),
            pl.BlockSpec((1, _ROWS_BLK, HID), lambda i: (1, i, 0)),
            pl.BlockSpec((IN_CH, HID), lambda i: (0, 0)),
            pl.BlockSpec((1, HID), lambda i: (0, 0)),
            pl.BlockSpec((HID, IN_CH), lambda i: (0, 0)),
            pl.BlockSpec((1, IN_CH), lambda i: (0, 0)),
        ],
        out_specs=pl.BlockSpec((_ROWS_BLK, IN_CH), lambda i: (i, 0)),
        out_shape=jax.ShapeDtypeStruct((N, IN_CH), jnp.float32),
    )(x, parts, parts, W_root, b_enc.reshape(1, HID), W_dec,
      b_dec.reshape(1, IN_CH))
    return out


def kernel(x, edge_index, W_root, W_rel, b_enc, W_dec, b_dec):
    return _forward(x, edge_index, W_root, W_rel, b_enc, W_dec, b_dec)


# speedup vs baseline: 1.1424x; 1.1424x over previous
import jax
import jax.numpy as jnp
from jax import lax
from jax.experimental import pallas as pl
from jax.experimental.pallas import tpu as pltpu
from jax.experimental.pallas import tpu_sc as plsc

N = 10000
E = 320000
IN_CH = 128
HID = 64

NC = 2
NS = 16
NW = NC * NS
CHUNK = 128
NCH = E // CHUNK
CPT = NCH // NW
TAIL = NCH - NW * CPT
NPAD = 10240
SLICE = NPAD // NS


def _enc_body(x_ref, w_ref, out_ref):
    out_ref[...] = jnp.dot(x_ref[...].astype(jnp.bfloat16),
                           w_ref[...].astype(jnp.bfloat16),
                           preferred_element_type=jnp.float32)


def _sc_body(gidx_ref, sidx_ref, y_ref, out_ref,
             gidx, sidx, rows0, rows1, rows2, rows3, agg,
             sem0, sem1, sem2, sem3):
    c = lax.axis_index("c")
    s = lax.axis_index("s")
    w = s * NC + c
    zv = jnp.zeros((16,), jnp.float32)

    def zrow(i, carry):
        for jj in range(HID // 16):
            rows0[i, pl.ds(jj * 16, 16)] = zv
        return carry

    lax.fori_loop(0, CHUNK, zrow, 0)
    for piece in range(SLICE // CHUNK):
        pltpu.sync_copy(
            rows0, agg.at[pl.ds(s * SLICE + piece * CHUNK, CHUNK)])
    base = CPT * w + jnp.minimum(w, TAIL)
    pltpu.sync_copy(gidx_ref.at[pl.ds(base, CPT)], gidx.at[pl.ds(0, CPT)])
    pltpu.sync_copy(sidx_ref.at[pl.ds(base, CPT)], sidx.at[pl.ds(0, CPT)])

    @pl.when(w < TAIL)
    def _():
        pltpu.sync_copy(gidx_ref.at[pl.ds(base + CPT, 1)],
                        gidx.at[pl.ds(CPT, 1)])
        pltpu.sync_copy(sidx_ref.at[pl.ds(base + CPT, 1)],
                        sidx.at[pl.ds(CPT, 1)])

    plsc.subcore_barrier()

    bufs = ((rows0, sem0), (rows1, sem1), (rows2, sem2), (rows3, sem3))
    pltpu.async_copy(y_ref.at[gidx.at[0]], rows0, sem0)
    pltpu.async_copy(y_ref.at[gidx.at[1]], rows1, sem1)
    pltpu.async_copy(y_ref.at[gidx.at[2]], rows2, sem2)
    last = CPT - 1

    def step(jj, carry):
        j = 4 * jj
        for k in range(4):
            fb, fs = bufs[(k + 3) % 4]
            wb, ws = bufs[k]
            jn = jnp.minimum(j + k + 3, last)
            pltpu.async_copy(y_ref.at[gidx.at[jn]], fb, fs)
            pltpu.make_async_copy(y_ref.at[gidx.at[j]], wb, ws).wait()
            pltpu.sync_copy(wb, agg.at[sidx.at[j + k]], add=True)
        return carry

    lax.fori_loop(0, CPT // 4, step, 0)
    pltpu.make_async_copy(y_ref.at[gidx.at[0]], rows0, sem0).wait()
    pltpu.sync_copy(rows0, agg.at[sidx.at[CPT - 2]], add=True)
    pltpu.make_async_copy(y_ref.at[gidx.at[0]], rows1, sem1).wait()
    pltpu.sync_copy(rows1, agg.at[sidx.at[CPT - 1]], add=True)
    pltpu.make_async_copy(y_ref.at[gidx.at[0]], rows2, sem2).wait()

    @pl.when(w < TAIL)
    def _():
        pltpu.async_copy(y_ref.at[gidx.at[CPT]], rows0, sem0)
        pltpu.make_async_copy(y_ref.at[gidx.at[CPT]], rows0, sem0).wait()
        pltpu.sync_copy(rows0, agg.at[sidx.at[CPT]], add=True)

    plsc.subcore_barrier()
    pltpu.sync_copy(agg.at[pl.ds(s * SLICE, SLICE)],
                    out_ref.at[c, pl.ds(s * SLICE, SLICE)])


_sc_scatter = pl.kernel(
    _sc_body,
    out_type=jax.ShapeDtypeStruct((NC, NPAD, HID), jnp.float32),
    mesh=plsc.VectorSubcoreMesh(core_axis_name="c", subcore_axis_name="s"),
    compiler_params=pltpu.CompilerParams(use_tc_tiling_on_sc=False),
    scratch_types=[
        pltpu.VMEM((CPT + 1, CHUNK), jnp.int32),
        pltpu.VMEM((CPT + 1, CHUNK), jnp.int32),
        pltpu.VMEM((CHUNK, HID), jnp.float32),
        pltpu.VMEM((CHUNK, HID), jnp.float32),
        pltpu.VMEM((CHUNK, HID), jnp.float32),
        pltpu.VMEM((CHUNK, HID), jnp.float32),
        pltpu.VMEM_SHARED((NPAD, HID), jnp.float32),
        pltpu.SemaphoreType.DMA,
        pltpu.SemaphoreType.DMA,
        pltpu.SemaphoreType.DMA,
        pltpu.SemaphoreType.DMA,
    ],
)


def _dec_body(x_ref, p0_ref, p1_ref, wroot_ref, benc_ref,
              wdec_ref, bdec_ref, out_ref):
    agg = p0_ref[0] + p1_ref[0]
    h = (
        jnp.dot(x_ref[...].astype(jnp.bfloat16),
                wroot_ref[...].astype(jnp.bfloat16),
                preferred_element_type=jnp.float32)
        + agg + benc_ref[...]
    )
    h = jnp.maximum(h, 0.0)
    out_ref[...] = (
        jnp.dot(h.astype(jnp.bfloat16), wdec_ref[...].astype(jnp.bfloat16),
                preferred_element_type=jnp.float32)
        + bdec_ref[...]
    )


_ROWS_BLK = 1280


@jax.jit
def _forward(x, edge_index, W_root, W_rel, b_enc, W_dec, b_dec):
    gidx = edge_index[0].astype(jnp.int32).reshape(NCH, CHUNK)
    sidx = edge_index[1].astype(jnp.int32).reshape(NCH, CHUNK)

    y = pl.pallas_call(
        _enc_body,
        grid=(NS,),
        in_specs=[
            pl.BlockSpec((SLICE, IN_CH), lambda i: (i, 0)),
            pl.BlockSpec((IN_CH, HID), lambda i: (0, 0)),
        ],
        out_specs=pl.BlockSpec((SLICE, HID), lambda i: (i, 0)),
        out_shape=jax.ShapeDtypeStruct((NPAD, HID), jnp.float32),
    )(x, W_rel)

    parts = _sc_scatter(gidx, sidx, y)

    out = pl.pallas_call(
        _dec_body,
        grid=(NPAD // _ROWS_BLK,),
        in_specs=[
            pl.BlockSpec((_ROWS_BLK, IN_CH), lambda i: (i, 0)),
            pl.BlockSpec((1, _ROWS_BLK, HID), lambda i: (0, i, 0)),
            pl.BlockSpec((1, _ROWS_BLK, HID), lambda i: (1, i, 0)),
            pl.BlockSpec((IN_CH, HID), lambda i: (0, 0)),
            pl.BlockSpec((1, HID), lambda i: (0, 0)),
            pl.BlockSpec((HID, IN_CH), lambda i: (0, 0)),
            pl.BlockSpec((1, IN_CH), lambda i: (0, 0)),
        ],
        out_specs=pl.BlockSpec((_ROWS_BLK, IN_CH), lambda i: (i, 0)),
        out_shape=jax.ShapeDtypeStruct((N, IN_CH), jnp.float32),
    )(x, parts, parts, W_root, b_enc.reshape(1, HID), W_dec,
      b_dec.reshape(1, IN_CH))
    return out


def kernel(x, edge_index, W_root, W_rel, b_enc, W_dec, b_dec):
    return _forward(x, edge_index, W_root, W_rel, b_enc, W_dec, b_dec)


# speedup vs baseline: 1.2042x; 1.0541x over previous
import jax
import jax.numpy as jnp
from jax import lax
from jax.experimental import pallas as pl
from jax.experimental.pallas import tpu as pltpu
from jax.experimental.pallas import tpu_sc as plsc

N = 10000
E = 320000
IN_CH = 128
HID = 64

NC = 2
NS = 16
NW = NC * NS
CHUNK = 128
NCH = E // CHUNK
CPT = NCH // NW
TAIL = NCH - NW * CPT
NPAD = 10240
SLICE = NPAD // NS


def _enc_body(x_ref, w_ref, out_ref):
    out_ref[...] = jnp.dot(x_ref[...].astype(jnp.bfloat16),
                           w_ref[...].astype(jnp.bfloat16),
                           preferred_element_type=jnp.float32)


def _sc_body(gidx_ref, sidx_ref, y_ref, out_ref,
             gidx, sidx, rows0, rows1, rows2, rows3, agg,
             sem0, sem1, sem2, sem3):
    c = lax.axis_index("c")
    s = lax.axis_index("s")
    w = s * NC + c
    zv = jnp.zeros((16,), jnp.float32)

    def zrow(i, carry):
        for jj in range(HID // 16):
            rows0[i, pl.ds(jj * 16, 16)] = zv
        return carry

    lax.fori_loop(0, CHUNK, zrow, 0)
    for piece in range(SLICE // CHUNK):
        pltpu.sync_copy(
            rows0, agg.at[pl.ds(s * SLICE + piece * CHUNK, CHUNK)])
    base = CPT * w + jnp.minimum(w, TAIL)
    pltpu.sync_copy(gidx_ref.at[pl.ds(base, CPT)], gidx.at[pl.ds(0, CPT)])
    pltpu.sync_copy(sidx_ref.at[pl.ds(base, CPT)], sidx.at[pl.ds(0, CPT)])

    @pl.when(w < TAIL)
    def _():
        pltpu.sync_copy(gidx_ref.at[pl.ds(base + CPT, 1)],
                        gidx.at[pl.ds(CPT, 1)])
        pltpu.sync_copy(sidx_ref.at[pl.ds(base + CPT, 1)],
                        sidx.at[pl.ds(CPT, 1)])

    plsc.subcore_barrier()

    bufs = ((rows0, sem0), (rows1, sem1), (rows2, sem2), (rows3, sem3))
    pltpu.async_copy(y_ref.at[gidx.at[0]], rows0, sem0)
    pltpu.async_copy(y_ref.at[gidx.at[1]], rows1, sem1)
    pltpu.async_copy(y_ref.at[gidx.at[2]], rows2, sem2)
    last = CPT - 1

    def step(jj, carry):
        j = 4 * jj
        for k in range(4):
            fb, fs = bufs[(k + 3) % 4]
            wb, ws = bufs[k]
            jn = jnp.minimum(j + k + 3, last)
            pltpu.async_copy(y_ref.at[gidx.at[jn]], fb, fs)
            pltpu.make_async_copy(y_ref.at[gidx.at[j]], wb, ws).wait()
            pltpu.sync_copy(wb, agg.at[sidx.at[j + k]], add=True)
        return carry

    lax.fori_loop(0, CPT // 4, step, 0)
    pltpu.make_async_copy(y_ref.at[gidx.at[0]], rows0, sem0).wait()
    pltpu.sync_copy(rows0, agg.at[sidx.at[CPT - 2]], add=True)
    pltpu.make_async_copy(y_ref.at[gidx.at[0]], rows1, sem1).wait()
    pltpu.sync_copy(rows1, agg.at[sidx.at[CPT - 1]], add=True)
    pltpu.make_async_copy(y_ref.at[gidx.at[0]], rows2, sem2).wait()

    @pl.when(w < TAIL)
    def _():
        pltpu.async_copy(y_ref.at[gidx.at[CPT]], rows0, sem0)
        pltpu.make_async_copy(y_ref.at[gidx.at[CPT]], rows0, sem0).wait()
        pltpu.sync_copy(rows0, agg.at[sidx.at[CPT]], add=True)

    plsc.subcore_barrier()
    pltpu.sync_copy(agg.at[pl.ds(s * SLICE, SLICE)],
                    out_ref.at[c, pl.ds(s * SLICE, SLICE)])


_sc_scatter = pl.kernel(
    _sc_body,
    out_type=jax.ShapeDtypeStruct((NC, NPAD, HID), jnp.float32),
    mesh=plsc.VectorSubcoreMesh(core_axis_name="c", subcore_axis_name="s"),
    compiler_params=pltpu.CompilerParams(use_tc_tiling_on_sc=False),
    scratch_types=[
        pltpu.VMEM((CPT + 1, CHUNK), jnp.int32),
        pltpu.VMEM((CPT + 1, CHUNK), jnp.int32),
        pltpu.VMEM((CHUNK, HID), jnp.float32),
        pltpu.VMEM((CHUNK, HID), jnp.float32),
        pltpu.VMEM((CHUNK, HID), jnp.float32),
        pltpu.VMEM((CHUNK, HID), jnp.float32),
        pltpu.VMEM_SHARED((NPAD, HID), jnp.float32),
        pltpu.SemaphoreType.DMA,
        pltpu.SemaphoreType.DMA,
        pltpu.SemaphoreType.DMA,
        pltpu.SemaphoreType.DMA,
    ],
)


def _dec_body(x_ref, p0_ref, p1_ref, wroot_ref, benc_ref,
              wdec_ref, bdec_ref, out_ref):
    agg = p0_ref[0] + p1_ref[0]
    h = (
        jnp.dot(x_ref[...].astype(jnp.bfloat16),
                wroot_ref[...].astype(jnp.bfloat16),
                preferred_element_type=jnp.float32)
        + agg + benc_ref[...]
    )
    h = jnp.maximum(h, 0.0)
    out_ref[...] = (
        jnp.dot(h.astype(jnp.bfloat16), wdec_ref[...].astype(jnp.bfloat16),
                preferred_element_type=jnp.float32)
        + bdec_ref[...]
    )


_ROWS_BLK = 1280


@jax.jit
def _forward(x, edge_index, W_root, W_rel, b_enc, W_dec, b_dec):
    gidx = edge_index[0].astype(jnp.int32).reshape(NCH, CHUNK)
    sidx = edge_index[1].astype(jnp.int32).reshape(NCH, CHUNK)

    y = pl.pallas_call(
        _enc_body,
        grid=(4,),
        in_specs=[
            pl.BlockSpec((NPAD // 4, IN_CH), lambda i: (i, 0)),
            pl.BlockSpec((IN_CH, HID), lambda i: (0, 0)),
        ],
        out_specs=pl.BlockSpec((NPAD // 4, HID), lambda i: (i, 0)),
        out_shape=jax.ShapeDtypeStruct((NPAD, HID), jnp.float32),
    )(x, W_rel)

    parts = _sc_scatter(gidx, sidx, y)

    out = pl.pallas_call(
        _dec_body,
        grid=(NPAD // _ROWS_BLK,),
        in_specs=[
            pl.BlockSpec((_ROWS_BLK, IN_CH), lambda i: (i, 0)),
            pl.BlockSpec((1, _ROWS_BLK, HID), lambda i: (0, i, 0)),
            pl.BlockSpec((1, _ROWS_BLK, HID), lambda i: (1, i, 0)),
            pl.BlockSpec((IN_CH, HID), lambda i: (0, 0)),
            pl.BlockSpec((1, HID), lambda i: (0, 0)),
            pl.BlockSpec((HID, IN_CH), lambda i: (0, 0)),
            pl.BlockSpec((1, IN_CH), lambda i: (0, 0)),
        ],
        out_specs=pl.BlockSpec((_ROWS_BLK, IN_CH), lambda i: (i, 0)),
        out_shape=jax.ShapeDtypeStruct((N, IN_CH), jnp.float32),
    )(x, parts, parts, W_root, b_enc.reshape(1, HID), W_dec,
      b_dec.reshape(1, IN_CH))
    return out


def kernel(x, edge_index, W_root, W_rel, b_enc, W_dec, b_dec):
    return _forward(x, edge_index, W_root, W_rel, b_enc, W_dec, b_dec)


# speedup vs baseline: 1.2273x; 1.0192x over previous
import jax
import jax.numpy as jnp
from jax import lax
from jax.experimental import pallas as pl
from jax.experimental.pallas import tpu as pltpu
from jax.experimental.pallas import tpu_sc as plsc

N = 10000
E = 320000
IN_CH = 128
HID = 64

NC = 2
NS = 16
NW = NC * NS
CHUNK = 128
NCH = E // CHUNK
CPT = NCH // NW
TAIL = NCH - NW * CPT
NPAD = 10240
SLICE = NPAD // NS


def _enc_body(x_ref, w_ref, out_ref):
    out_ref[...] = jnp.dot(x_ref[...].astype(jnp.bfloat16),
                           w_ref[...].astype(jnp.bfloat16),
                           preferred_element_type=jnp.float32)


def _sc_body(gidx_ref, sidx_ref, y_ref, out_ref,
             gidx, sidx, rows0, rows1, rows2, rows3, agg,
             sem0, sem1, sem2, sem3):
    c = lax.axis_index("c")
    s = lax.axis_index("s")
    w = s * NC + c
    zv = jnp.zeros((16,), jnp.float32)

    def zrow(i, carry):
        for jj in range(HID // 16):
            rows0[i, pl.ds(jj * 16, 16)] = zv
        return carry

    lax.fori_loop(0, CHUNK, zrow, 0)
    for piece in range(SLICE // CHUNK):
        pltpu.sync_copy(
            rows0, agg.at[pl.ds(s * SLICE + piece * CHUNK, CHUNK)])
    base = CPT * w + jnp.minimum(w, TAIL)
    pltpu.sync_copy(gidx_ref.at[pl.ds(base, CPT)], gidx.at[pl.ds(0, CPT)])
    pltpu.sync_copy(sidx_ref.at[pl.ds(base, CPT)], sidx.at[pl.ds(0, CPT)])

    @pl.when(w < TAIL)
    def _():
        pltpu.sync_copy(gidx_ref.at[pl.ds(base + CPT, 1)],
                        gidx.at[pl.ds(CPT, 1)])
        pltpu.sync_copy(sidx_ref.at[pl.ds(base + CPT, 1)],
                        sidx.at[pl.ds(CPT, 1)])

    plsc.subcore_barrier()

    bufs = ((rows0, sem0), (rows1, sem1), (rows2, sem2), (rows3, sem3))
    pltpu.async_copy(y_ref.at[gidx.at[0]], rows0, sem0)
    pltpu.async_copy(y_ref.at[gidx.at[1]], rows1, sem1)
    pltpu.async_copy(y_ref.at[gidx.at[2]], rows2, sem2)
    last = CPT - 1

    def step(jj, carry):
        j = 4 * jj
        for k in range(4):
            fb, fs = bufs[(k + 3) % 4]
            wb, ws = bufs[k]
            jn = jnp.minimum(j + k + 3, last)
            pltpu.async_copy(y_ref.at[gidx.at[jn]], fb, fs)
            pltpu.make_async_copy(y_ref.at[gidx.at[j]], wb, ws).wait()
            pltpu.sync_copy(wb, agg.at[sidx.at[j + k]], add=True)
        return carry

    lax.fori_loop(0, CPT // 4, step, 0)
    pltpu.make_async_copy(y_ref.at[gidx.at[0]], rows0, sem0).wait()
    pltpu.sync_copy(rows0, agg.at[sidx.at[CPT - 2]], add=True)
    pltpu.make_async_copy(y_ref.at[gidx.at[0]], rows1, sem1).wait()
    pltpu.sync_copy(rows1, agg.at[sidx.at[CPT - 1]], add=True)
    pltpu.make_async_copy(y_ref.at[gidx.at[0]], rows2, sem2).wait()

    @pl.when(w < TAIL)
    def _():
        pltpu.async_copy(y_ref.at[gidx.at[CPT]], rows0, sem0)
        pltpu.make_async_copy(y_ref.at[gidx.at[CPT]], rows0, sem0).wait()
        pltpu.sync_copy(rows0, agg.at[sidx.at[CPT]], add=True)

    plsc.subcore_barrier()
    pltpu.sync_copy(agg.at[pl.ds(s * SLICE, SLICE)],
                    out_ref.at[c, pl.ds(s * SLICE, SLICE)])


_sc_scatter = pl.kernel(
    _sc_body,
    out_type=jax.ShapeDtypeStruct((NC, NPAD, HID), jnp.float32),
    mesh=plsc.VectorSubcoreMesh(core_axis_name="c", subcore_axis_name="s"),
    compiler_params=pltpu.CompilerParams(use_tc_tiling_on_sc=False),
    scratch_types=[
        pltpu.VMEM((CPT + 1, CHUNK), jnp.int32),
        pltpu.VMEM((CPT + 1, CHUNK), jnp.int32),
        pltpu.VMEM((CHUNK, HID), jnp.float32),
        pltpu.VMEM((CHUNK, HID), jnp.float32),
        pltpu.VMEM((CHUNK, HID), jnp.float32),
        pltpu.VMEM((CHUNK, HID), jnp.float32),
        pltpu.VMEM_SHARED((NPAD, HID), jnp.float32),
        pltpu.SemaphoreType.DMA,
        pltpu.SemaphoreType.DMA,
        pltpu.SemaphoreType.DMA,
        pltpu.SemaphoreType.DMA,
    ],
)


def _dec_body(x_ref, p0_ref, p1_ref, wroot_ref, benc_ref,
              wdec_ref, bdec_ref, out_ref):
    agg = p0_ref[0] + p1_ref[0]
    h = (
        jnp.dot(x_ref[...].astype(jnp.bfloat16),
                wroot_ref[...].astype(jnp.bfloat16),
                preferred_element_type=jnp.float32)
        + agg + benc_ref[...]
    )
    h = jnp.maximum(h, 0.0)
    out_ref[...] = (
        jnp.dot(h.astype(jnp.bfloat16), wdec_ref[...].astype(jnp.bfloat16),
                preferred_element_type=jnp.float32)
        + bdec_ref[...]
    )


_ROWS_BLK = 2560


@jax.jit
def _forward(x, edge_index, W_root, W_rel, b_enc, W_dec, b_dec):
    gidx = edge_index[0].astype(jnp.int32).reshape(NCH, CHUNK)
    sidx = edge_index[1].astype(jnp.int32).reshape(NCH, CHUNK)

    y = pl.pallas_call(
        _enc_body,
        grid=(4,),
        in_specs=[
            pl.BlockSpec((NPAD // 4, IN_CH), lambda i: (i, 0)),
            pl.BlockSpec((IN_CH, HID), lambda i: (0, 0)),
        ],
        out_specs=pl.BlockSpec((NPAD // 4, HID), lambda i: (i, 0)),
        out_shape=jax.ShapeDtypeStruct((NPAD, HID), jnp.float32),
    )(x, W_rel)

    parts = _sc_scatter(gidx, sidx, y)

    out = pl.pallas_call(
        _dec_body,
        grid=(NPAD // _ROWS_BLK,),
        in_specs=[
            pl.BlockSpec((_ROWS_BLK, IN_CH), lambda i: (i, 0)),
            pl.BlockSpec((1, _ROWS_BLK, HID), lambda i: (0, i, 0)),
            pl.BlockSpec((1, _ROWS_BLK, HID), lambda i: (1, i, 0)),
            pl.BlockSpec((IN_CH, HID), lambda i: (0, 0)),
            pl.BlockSpec((1, HID), lambda i: (0, 0)),
            pl.BlockSpec((HID, IN_CH), lambda i: (0, 0)),
            pl.BlockSpec((1, IN_CH), lambda i: (0, 0)),
        ],
        out_specs=pl.BlockSpec((_ROWS_BLK, IN_CH), lambda i: (i, 0)),
        out_shape=jax.ShapeDtypeStruct((N, IN_CH), jnp.float32),
    )(x, parts, parts, W_root, b_enc.reshape(1, HID), W_dec,
      b_dec.reshape(1, IN_CH))
    return out


def kernel(x, edge_index, W_root, W_rel, b_enc, W_dec, b_dec):
    return _forward(x, edge_index, W_root, W_rel, b_enc, W_dec, b_dec)
